# baseline (speedup 1.0000x reference)
import functools

import jax
import jax.numpy as jnp
from jax import lax
from jax.experimental import pallas as pl
from jax.experimental.pallas import tpu as pltpu
from jax.experimental.pallas import tpu_sc as plsc

NC = 2
NS = 16
LANES = 16
C = 80


def _mesh():
    return plsc.VectorSubcoreMesh(
        core_axis_name="c", subcore_axis_name="s", num_cores=NC, num_subcores=NS)


def _build_deg(E, NP):
    NT = NC * NS
    nchunks = (E // NT) // C
    rows_pt = NP // NS
    ZR = 128
    NZ = rows_pt // ZR

    @functools.partial(
        pl.kernel,
        out_type=jax.ShapeDtypeStruct((NC, NP, LANES), jnp.float32),
        mesh=_mesh(),
        scratch_types=[
            pltpu.VMEM((nchunks, C), jnp.int32),
            pltpu.VMEM((C, LANES), jnp.float32),
            pltpu.VMEM((ZR, LANES), jnp.float32),
            pltpu.VMEM_SHARED((NP, LANES), jnp.float32),
        ],
        compiler_params=pltpu.CompilerParams(use_tc_tiling_on_sc=False),
    )
    def deg_kernel(dst_hbm, out_hbm, didx, ones_v, zb, sh):
        cid = lax.axis_index("c")
        sid = lax.axis_index("s")
        wid = cid * NS + sid

        def fill(i, _):
            ones_v[i] = jnp.full((LANES,), 1.0, jnp.float32)
            return 0
        lax.fori_loop(0, C, fill, 0)

        def fillz(i, _):
            zb[i, :] = jnp.zeros((LANES,), jnp.float32)
            return 0
        lax.fori_loop(0, ZR, fillz, 0)

        base_row = sid * rows_pt
        for j in range(NZ):
            pltpu.sync_copy(zb, sh.at[pl.ds(base_row + j * ZR, ZR)])
        pltpu.sync_copy(dst_hbm.at[wid], didx)
        plsc.subcore_barrier()

        def step(k, _):
            pltpu.sync_copy(ones_v, sh.at[didx.at[k]], add=True)
            return 0
        lax.fori_loop(0, nchunks, step, 0)
        plsc.subcore_barrier()
        pltpu.sync_copy(sh.at[pl.ds(base_row, rows_pt)],
                        out_hbm.at[cid, pl.ds(base_row, rows_pt)])

    return deg_kernel


def _build_scatter(NP, HD, nchunks):
    HH = HD // NC
    CC = 128
    NBUF = 4
    L = 2
    rows_pt = NP // NS
    ZR = 128
    NZ = rows_pt // ZR
    nrings = nchunks // NBUF

    @functools.partial(
        pl.kernel,
        out_type=jax.ShapeDtypeStruct((NC, NP, HH), jnp.float32),
        mesh=_mesh(),
        scratch_types=[
            pltpu.VMEM((nchunks, CC), jnp.int32),
            pltpu.VMEM((nchunks, CC), jnp.int32),
            pltpu.VMEM((NBUF, CC, HH), jnp.float32),
            pltpu.VMEM((ZR, HH), jnp.float32),
            pltpu.VMEM_SHARED((NP, HH), jnp.float32),
            pltpu.SemaphoreType.DMA((NBUF,)),
            pltpu.SemaphoreType.DMA((NBUF,)),
        ],
        compiler_params=pltpu.CompilerParams(use_tc_tiling_on_sc=False),
    )
    def scat_kernel(hwp_hbm, src_hbm, dst_hbm, out_hbm, sidx, didx, rows, zb, sh, gsem, ssem):
        cid = lax.axis_index("c")
        sid = lax.axis_index("s")
        half_rows = hwp_hbm.shape[0] // NC

        def fillz(i, _):
            for j in range(HH // LANES):
                zb[i, pl.ds(j * LANES, LANES)] = jnp.zeros((LANES,), jnp.float32)
            return 0
        lax.fori_loop(0, ZR, fillz, 0)

        base_row = sid * rows_pt
        for j in range(NZ):
            pltpu.sync_copy(zb, sh.at[pl.ds(base_row + j * ZR, ZR)])
        pltpu.sync_copy(src_hbm.at[sid], sidx)
        pltpu.sync_copy(dst_hbm.at[sid], didx)

        off = cid * half_rows

        def adj(k, _):
            for j in range(CC // LANES):
                sl = pl.ds(j * LANES, LANES)
                sidx[k, sl] = sidx[k, sl] + off
            return 0
        lax.fori_loop(0, nchunks, adj, 0)
        plsc.subcore_barrier()

        def gissue(k, b):
            pltpu.async_copy(hwp_hbm.at[sidx.at[k]], rows.at[b], gsem.at[b])

        def gwait(b):
            pltpu.make_async_copy(hwp_hbm.at[sidx.at[0]], rows.at[b], gsem.at[b]).wait()

        def sissue(k, b):
            pltpu.async_copy(rows.at[b], sh.at[didx.at[k]], ssem.at[b], add=True)

        def swait(b):
            pltpu.make_async_copy(rows.at[b], sh.at[didx.at[0]], ssem.at[b]).wait()

        def step(k, _):
            pltpu.async_copy(hwp_hbm.at[sidx.at[k]], rows.at[0], gsem.at[0]).wait()
            pltpu.sync_copy(rows.at[0], sh.at[didx.at[k]], add=True)
            return 0
        lax.fori_loop(0, nchunks, step, 0)

        plsc.subcore_barrier()
        pltpu.sync_copy(sh.at[pl.ds(base_row, rows_pt)],
                        out_hbm.at[cid, pl.ds(base_row, rows_pt)])

    return scat_kernel


def _build_pool(N, HD, B):
    NT = NC * NS
    P = 320

    @functools.partial(
        pl.kernel,
        out_type=jax.ShapeDtypeStruct((NT, B, HD), jnp.float32),
        mesh=_mesh(),
        scratch_types=[
            pltpu.VMEM((P, HD), jnp.float32),
            pltpu.VMEM((P,), jnp.int32),
            pltpu.VMEM((B, HD), jnp.float32),
        ],
    )
    def pool_kernel(h_hbm, batch_hbm, out_hbm, hv, bv, acc):
        cid = lax.axis_index("c")
        sid = lax.axis_index("s")
        wid = cid * NS + sid
        base = jnp.minimum(wid * P, N - P)

        def filln(i, _):
            for j in range(HD // LANES):
                acc[i, pl.ds(j * LANES, LANES)] = jnp.full(
                    (LANES,), -jnp.inf, jnp.float32)
            return 0
        lax.fori_loop(0, B, filln, 0)

        pltpu.sync_copy(h_hbm.at[pl.ds(base, P)], hv)
        pltpu.sync_copy(batch_hbm.at[pl.ds(base, P)], bv)

        def group(gi, _):
            bvec = bv[pl.ds(gi * LANES, LANES)]
            for l in range(LANES):
                b = bvec[l]
                for j in range(HD // LANES):
                    sl = pl.ds(j * LANES, LANES)
                    acc[b, sl] = jnp.maximum(acc[b, sl], hv[gi * LANES + l, sl])
            return 0
        lax.fori_loop(0, P // LANES, group, 0)
        pltpu.sync_copy(acc, out_hbm.at[wid])

    return pool_kernel


def _k0(x, W, degp, RB=1000):
    N, D = x.shape
    HD = W.shape[1]
    HH = HD // NC

    def body(x_ref, w_ref, degp_ref, hwp_ref, dis_ref):
        deg = degp_ref[0, :, 0:1] + degp_ref[1, :, 0:1] + 1.0
        dis = jnp.where(deg > 0, lax.rsqrt(jnp.maximum(deg, 1e-12)), 0.0)
        hw = jnp.dot(x_ref[...], w_ref[...], preferred_element_type=jnp.float32)
        hwp = hw * dis
        hwp_ref[0] = hwp[:, :HH]
        hwp_ref[1] = hwp[:, HH:]
        dis_ref[...] = dis

    return pl.pallas_call(
        body,
        grid=(N // RB,),
        in_specs=[
            pl.BlockSpec((RB, D), lambda i: (i, 0)),
            pl.BlockSpec((D, HD), lambda i: (0, 0)),
            pl.BlockSpec((NC, RB, LANES), lambda i: (0, i, 0)),
        ],
        out_specs=[
            pl.BlockSpec((NC, RB, HH), lambda i: (0, i, 0)),
            pl.BlockSpec((RB, 1), lambda i: (i, 0)),
        ],
        out_shape=[
            jax.ShapeDtypeStruct((NC, N, HH), jnp.float32),
            jax.ShapeDtypeStruct((N, 1), jnp.float32),
        ],
    )(x, W, degp)


def _kc(parts, hwp, dis, b, W, RB=1000):
    NCp, N, HH = hwp.shape
    HD = W.shape[0]

    def body(part_ref, hwp_ref, dis_ref, b_ref, w_ref, out_ref):
        s = jnp.concatenate(
            [part_ref[0] + hwp_ref[0], part_ref[1] + hwp_ref[1]], axis=1)
        h = jnp.maximum(s * dis_ref[...] + b_ref[...], 0.0)
        o = jnp.dot(h, w_ref[...], preferred_element_type=jnp.float32) * dis_ref[...]
        out_ref[0] = o[:, :HH]
        out_ref[1] = o[:, HH:]

    return pl.pallas_call(
        body,
        grid=(N // RB,),
        in_specs=[
            pl.BlockSpec((NC, RB, HH), lambda i: (0, i, 0)),
            pl.BlockSpec((NC, RB, HH), lambda i: (0, i, 0)),
            pl.BlockSpec((RB, 1), lambda i: (i, 0)),
            pl.BlockSpec((1, HD), lambda i: (0, 0)),
            pl.BlockSpec((HD, HD), lambda i: (0, 0)),
        ],
        out_specs=pl.BlockSpec((NC, RB, HH), lambda i: (0, i, 0)),
        out_shape=jax.ShapeDtypeStruct((NC, N, HH), jnp.float32),
    )(parts, hwp, dis, b, W)


def _kc_final(parts, hwp, dis, b, RB=1000):
    NCp, N, HH = hwp.shape
    HD = NCp * HH

    def body(part_ref, hwp_ref, dis_ref, b_ref, out_ref):
        s = jnp.concatenate(
            [part_ref[0] + hwp_ref[0], part_ref[1] + hwp_ref[1]], axis=1)
        out_ref[...] = jnp.maximum(s * dis_ref[...] + b_ref[...], 0.0)

    return pl.pallas_call(
        body,
        grid=(N // RB,),
        in_specs=[
            pl.BlockSpec((NC, RB, HH), lambda i: (0, i, 0)),
            pl.BlockSpec((NC, RB, HH), lambda i: (0, i, 0)),
            pl.BlockSpec((RB, 1), lambda i: (i, 0)),
            pl.BlockSpec((1, HD), lambda i: (0, 0)),
        ],
        out_specs=pl.BlockSpec((RB, HD), lambda i: (i, 0)),
        out_shape=jax.ShapeDtypeStruct((N, HD), jnp.float32),
    )(parts, hwp, dis, b)


def _head(gp, fr, ad, W_r1, b_r1, W_r2, b_r2, W_out, b_out):
    B = fr.shape[0]
    PROP = W_out.shape[1]

    def body(gp_ref, fr_ref, ad_ref, wr1, br1, wr2, br2, wo, bo, out_ref):
        g = jnp.max(gp_ref[...], axis=0)
        z = jnp.concatenate([g, fr_ref[...], ad_ref[...]], axis=1)
        z1 = jnp.dot(z, wr1[...], preferred_element_type=jnp.float32) + br1[...]
        s = z1 * jax.nn.sigmoid(z1)
        z = z + jnp.dot(s, wr2[...], preferred_element_type=jnp.float32) + br2[...]
        out_ref[...] = jnp.dot(
            z, wo[...], preferred_element_type=jnp.float32) + bo[...]

    return pl.pallas_call(
        body,
        out_shape=jax.ShapeDtypeStruct((B, PROP), jnp.float32),
    )(gp, fr, ad, W_r1, b_r1, W_r2, b_r2, W_out, b_out)


def kernel(x, edge_index, batch, frag_levels, adduct_feats,
           W_in, b_in, W_mid, b_mid, W_r1, b_r1, W_r2, b_r2, W_out, b_out):
    N, D = x.shape
    HD = W_in.shape[1]
    E = edge_index.shape[1]
    B = frag_levels.shape[0] // 8
    NT = NC * NS

    src32 = edge_index[0].astype(jnp.int32)
    dst32 = edge_index[1].astype(jnp.int32)
    dst_r32 = dst32.reshape(NT, (E // NT) // C, C)
    CC = 128
    nchunks = -(-(E // NS) // (CC * 4)) * 4
    Epad = NS * nchunks * CC
    src_r16 = jnp.concatenate(
        [src32, jnp.zeros((Epad - E,), jnp.int32)]).reshape(NS, nchunks, CC)
    dst_r16 = jnp.concatenate(
        [dst32, jnp.full((Epad - E,), N + 16, jnp.int32)]).reshape(NS, nchunks, CC)
    batch32 = batch.astype(jnp.int32)
    fr = frag_levels.reshape(B, 8)
    ad = adduct_feats.reshape(B, 8)

    NP = NS * 640
    deg_call = _build_deg(E, NP)
    scat_call = _build_scatter(NP, HD, nchunks)
    pool_call = _build_pool(N, HD, B)

    degp = deg_call(dst_r32)
    hwp, dis = _k0(x, W_in, degp)

    biases = [b_in.reshape(1, HD)] + [b_mid[i].reshape(1, HD) for i in range(W_mid.shape[0])]
    Ws = [W_mid[i] for i in range(W_mid.shape[0])]
    nlayers = 1 + W_mid.shape[0]

    h = None
    for li in range(nlayers):
        parts = scat_call(hwp.reshape(NC * N, HD // NC), src_r16, dst_r16)
        if li < nlayers - 1:
            hwp = _kc(parts, hwp, dis, biases[li], Ws[li])
        else:
            h = _kc_final(parts, hwp, dis, biases[li])

    gp = pool_call(h, batch32)
    return _head(gp, fr, ad, W_r1, b_r1.reshape(1, HD), W_r2,
                 b_r2.reshape(1, HD + 16), W_out, b_out.reshape(1, W_out.shape[1]))


# speedup vs baseline: 1.6601x; 1.6601x over previous
import functools

import jax
import jax.numpy as jnp
from jax import lax
from jax.experimental import pallas as pl
from jax.experimental.pallas import tpu as pltpu
from jax.experimental.pallas import tpu_sc as plsc

NC = 2
NS = 16
LANES = 16
C = 80


def _mesh():
    return plsc.VectorSubcoreMesh(
        core_axis_name="c", subcore_axis_name="s", num_cores=NC, num_subcores=NS)


def _build_deg(E, NP):
    NT = NC * NS
    nchunks = (E // NT) // C
    rows_pt = NP // NS
    ZR = 128
    NZ = rows_pt // ZR

    @functools.partial(
        pl.kernel,
        out_type=jax.ShapeDtypeStruct((NC, NP, LANES), jnp.float32),
        mesh=_mesh(),
        scratch_types=[
            pltpu.VMEM((nchunks, C), jnp.int32),
            pltpu.VMEM((C, LANES), jnp.float32),
            pltpu.VMEM((ZR, LANES), jnp.float32),
            pltpu.VMEM_SHARED((NP, LANES), jnp.float32),
        ],
        compiler_params=pltpu.CompilerParams(use_tc_tiling_on_sc=False),
    )
    def deg_kernel(dst_hbm, out_hbm, didx, ones_v, zb, sh):
        cid = lax.axis_index("c")
        sid = lax.axis_index("s")
        wid = cid * NS + sid

        def fill(i, _):
            ones_v[i] = jnp.full((LANES,), 1.0, jnp.float32)
            return 0
        lax.fori_loop(0, C, fill, 0)

        def fillz(i, _):
            zb[i, :] = jnp.zeros((LANES,), jnp.float32)
            return 0
        lax.fori_loop(0, ZR, fillz, 0)

        base_row = sid * rows_pt
        for j in range(NZ):
            pltpu.sync_copy(zb, sh.at[pl.ds(base_row + j * ZR, ZR)])
        pltpu.sync_copy(dst_hbm.at[wid], didx)
        plsc.subcore_barrier()

        def step(k, _):
            pltpu.sync_copy(ones_v, sh.at[didx.at[k]], add=True)
            return 0
        lax.fori_loop(0, nchunks, step, 0)
        plsc.subcore_barrier()
        pltpu.sync_copy(sh.at[pl.ds(base_row, rows_pt)],
                        out_hbm.at[cid, pl.ds(base_row, rows_pt)])

    return deg_kernel


def _build_scatter(NP, HD, nchunks):
    HH = HD // NC
    CC = 128
    NBUF = 4
    L = 2
    rows_pt = NP // NS
    ZR = 128
    NZ = rows_pt // ZR
    nrings = nchunks // NBUF

    @functools.partial(
        pl.kernel,
        out_type=jax.ShapeDtypeStruct((NC, NP, HH), jnp.float32),
        mesh=_mesh(),
        scratch_types=[
            pltpu.VMEM((nchunks, CC), jnp.int32),
            pltpu.VMEM((nchunks, CC), jnp.int32),
            pltpu.VMEM((NBUF, CC, HH), jnp.float32),
            pltpu.VMEM((ZR, HH), jnp.float32),
            pltpu.VMEM_SHARED((NP, HH), jnp.float32),
            pltpu.SemaphoreType.DMA((NBUF,)),
            pltpu.SemaphoreType.DMA((NBUF,)),
        ],
        compiler_params=pltpu.CompilerParams(use_tc_tiling_on_sc=False),
    )
    def scat_kernel(hwp_hbm, src_hbm, dst_hbm, out_hbm, sidx, didx, rows, zb, sh, gsem, ssem):
        cid = lax.axis_index("c")
        sid = lax.axis_index("s")
        half_rows = hwp_hbm.shape[0] // NC

        def fillz(i, _):
            for j in range(HH // LANES):
                zb[i, pl.ds(j * LANES, LANES)] = jnp.zeros((LANES,), jnp.float32)
            return 0
        lax.fori_loop(0, ZR, fillz, 0)

        base_row = sid * rows_pt
        for j in range(NZ):
            pltpu.sync_copy(zb, sh.at[pl.ds(base_row + j * ZR, ZR)])
        pltpu.sync_copy(src_hbm.at[sid], sidx)
        pltpu.sync_copy(dst_hbm.at[sid], didx)

        off = cid * half_rows

        def adj(k, _):
            for j in range(CC // LANES):
                sl = pl.ds(j * LANES, LANES)
                sidx[k, sl] = sidx[k, sl] + off
            return 0
        lax.fori_loop(0, nchunks, adj, 0)
        plsc.subcore_barrier()

        def gissue(k, b):
            pltpu.async_copy(hwp_hbm.at[sidx.at[k]], rows.at[b], gsem.at[b])

        def gwait(b):
            pltpu.make_async_copy(hwp_hbm.at[sidx.at[0]], rows.at[b], gsem.at[b]).wait()

        def sissue(k, b):
            pltpu.async_copy(rows.at[b], sh.at[didx.at[k]], ssem.at[b], add=True)

        def swait(b):
            pltpu.make_async_copy(rows.at[b], sh.at[didx.at[0]], ssem.at[b]).wait()

        for b in range(L):
            gissue(b, b)
        for b in range(NBUF):
            kn = b + L
            if kn >= NBUF:
                swait(kn % NBUF)
            gissue(kn, kn % NBUF)
            gwait(b)
            sissue(b, b)

        def ring(r, _):
            for b in range(NBUF):
                k = r * NBUF + b
                bn = (b + L) % NBUF
                swait(bn)
                gissue(k + L, bn)
                gwait(b)
                sissue(k, b)
            return 0
        lax.fori_loop(1, nrings - 1, ring, 0)

        for b in range(NBUF):
            k = (nrings - 1) * NBUF + b
            kn = k + L
            bn = (b + L) % NBUF
            if kn < nchunks:
                swait(bn)
                gissue(kn, bn)
            gwait(b)
            sissue(k, b)
        for b in range(NBUF):
            swait(b)

        plsc.subcore_barrier()
        pltpu.sync_copy(sh.at[pl.ds(base_row, rows_pt)],
                        out_hbm.at[cid, pl.ds(base_row, rows_pt)])

    return scat_kernel


def _build_pool(N, HD, B):
    NT = NC * NS
    P = 320

    @functools.partial(
        pl.kernel,
        out_type=jax.ShapeDtypeStruct((NT, B, HD), jnp.float32),
        mesh=_mesh(),
        scratch_types=[
            pltpu.VMEM((P, HD), jnp.float32),
            pltpu.VMEM((P,), jnp.int32),
            pltpu.VMEM((B, HD), jnp.float32),
        ],
    )
    def pool_kernel(h_hbm, batch_hbm, out_hbm, hv, bv, acc):
        cid = lax.axis_index("c")
        sid = lax.axis_index("s")
        wid = cid * NS + sid
        base = jnp.minimum(wid * P, N - P)

        def filln(i, _):
            for j in range(HD // LANES):
                acc[i, pl.ds(j * LANES, LANES)] = jnp.full(
                    (LANES,), -jnp.inf, jnp.float32)
            return 0
        lax.fori_loop(0, B, filln, 0)

        pltpu.sync_copy(h_hbm.at[pl.ds(base, P)], hv)
        pltpu.sync_copy(batch_hbm.at[pl.ds(base, P)], bv)

        def group(gi, _):
            bvec = bv[pl.ds(gi * LANES, LANES)]
            for l in range(LANES):
                b = bvec[l]
                for j in range(HD // LANES):
                    sl = pl.ds(j * LANES, LANES)
                    acc[b, sl] = jnp.maximum(acc[b, sl], hv[gi * LANES + l, sl])
            return 0
        lax.fori_loop(0, P // LANES, group, 0)
        pltpu.sync_copy(acc, out_hbm.at[wid])

    return pool_kernel


def _k0(x, W, degp, RB=1000):
    N, D = x.shape
    HD = W.shape[1]
    HH = HD // NC

    def body(x_ref, w_ref, degp_ref, hwp_ref, dis_ref):
        deg = degp_ref[0, :, 0:1] + degp_ref[1, :, 0:1] + 1.0
        dis = jnp.where(deg > 0, lax.rsqrt(jnp.maximum(deg, 1e-12)), 0.0)
        hw = jnp.dot(x_ref[...], w_ref[...], preferred_element_type=jnp.float32)
        hwp = hw * dis
        hwp_ref[0] = hwp[:, :HH]
        hwp_ref[1] = hwp[:, HH:]
        dis_ref[...] = dis

    return pl.pallas_call(
        body,
        grid=(N // RB,),
        in_specs=[
            pl.BlockSpec((RB, D), lambda i: (i, 0)),
            pl.BlockSpec((D, HD), lambda i: (0, 0)),
            pl.BlockSpec((NC, RB, LANES), lambda i: (0, i, 0)),
        ],
        out_specs=[
            pl.BlockSpec((NC, RB, HH), lambda i: (0, i, 0)),
            pl.BlockSpec((RB, 1), lambda i: (i, 0)),
        ],
        out_shape=[
            jax.ShapeDtypeStruct((NC, N, HH), jnp.float32),
            jax.ShapeDtypeStruct((N, 1), jnp.float32),
        ],
    )(x, W, degp)


def _kc(parts, hwp, dis, b, W, RB=1000):
    NCp, N, HH = hwp.shape
    HD = W.shape[0]

    def body(part_ref, hwp_ref, dis_ref, b_ref, w_ref, out_ref):
        s = jnp.concatenate(
            [part_ref[0] + hwp_ref[0], part_ref[1] + hwp_ref[1]], axis=1)
        h = jnp.maximum(s * dis_ref[...] + b_ref[...], 0.0)
        o = jnp.dot(h, w_ref[...], preferred_element_type=jnp.float32) * dis_ref[...]
        out_ref[0] = o[:, :HH]
        out_ref[1] = o[:, HH:]

    return pl.pallas_call(
        body,
        grid=(N // RB,),
        in_specs=[
            pl.BlockSpec((NC, RB, HH), lambda i: (0, i, 0)),
            pl.BlockSpec((NC, RB, HH), lambda i: (0, i, 0)),
            pl.BlockSpec((RB, 1), lambda i: (i, 0)),
            pl.BlockSpec((1, HD), lambda i: (0, 0)),
            pl.BlockSpec((HD, HD), lambda i: (0, 0)),
        ],
        out_specs=pl.BlockSpec((NC, RB, HH), lambda i: (0, i, 0)),
        out_shape=jax.ShapeDtypeStruct((NC, N, HH), jnp.float32),
    )(parts, hwp, dis, b, W)


def _kc_final(parts, hwp, dis, b, RB=1000):
    NCp, N, HH = hwp.shape
    HD = NCp * HH

    def body(part_ref, hwp_ref, dis_ref, b_ref, out_ref):
        s = jnp.concatenate(
            [part_ref[0] + hwp_ref[0], part_ref[1] + hwp_ref[1]], axis=1)
        out_ref[...] = jnp.maximum(s * dis_ref[...] + b_ref[...], 0.0)

    return pl.pallas_call(
        body,
        grid=(N // RB,),
        in_specs=[
            pl.BlockSpec((NC, RB, HH), lambda i: (0, i, 0)),
            pl.BlockSpec((NC, RB, HH), lambda i: (0, i, 0)),
            pl.BlockSpec((RB, 1), lambda i: (i, 0)),
            pl.BlockSpec((1, HD), lambda i: (0, 0)),
        ],
        out_specs=pl.BlockSpec((RB, HD), lambda i: (i, 0)),
        out_shape=jax.ShapeDtypeStruct((N, HD), jnp.float32),
    )(parts, hwp, dis, b)


def _head(gp, fr, ad, W_r1, b_r1, W_r2, b_r2, W_out, b_out):
    B = fr.shape[0]
    PROP = W_out.shape[1]

    def body(gp_ref, fr_ref, ad_ref, wr1, br1, wr2, br2, wo, bo, out_ref):
        g = jnp.max(gp_ref[...], axis=0)
        z = jnp.concatenate([g, fr_ref[...], ad_ref[...]], axis=1)
        z1 = jnp.dot(z, wr1[...], preferred_element_type=jnp.float32) + br1[...]
        s = z1 * jax.nn.sigmoid(z1)
        z = z + jnp.dot(s, wr2[...], preferred_element_type=jnp.float32) + br2[...]
        out_ref[...] = jnp.dot(
            z, wo[...], preferred_element_type=jnp.float32) + bo[...]

    return pl.pallas_call(
        body,
        out_shape=jax.ShapeDtypeStruct((B, PROP), jnp.float32),
    )(gp, fr, ad, W_r1, b_r1, W_r2, b_r2, W_out, b_out)


def kernel(x, edge_index, batch, frag_levels, adduct_feats,
           W_in, b_in, W_mid, b_mid, W_r1, b_r1, W_r2, b_r2, W_out, b_out):
    N, D = x.shape
    HD = W_in.shape[1]
    E = edge_index.shape[1]
    B = frag_levels.shape[0] // 8
    NT = NC * NS

    src32 = edge_index[0].astype(jnp.int32)
    dst32 = edge_index[1].astype(jnp.int32)
    dst_r32 = dst32.reshape(NT, (E // NT) // C, C)
    CC = 128
    nchunks = -(-(E // NS) // (CC * 4)) * 4
    ept = E // NS
    npad = nchunks * CC - ept
    src_pad = jnp.zeros((NS, npad), jnp.int32)
    dst_pad = jnp.broadcast_to(N + (jnp.arange(npad, dtype=jnp.int32) % 240),
                               (NS, npad))
    src_r16 = jnp.concatenate(
        [src32.reshape(NS, ept), src_pad], axis=1).reshape(NS, nchunks, CC)
    dst_r16 = jnp.concatenate(
        [dst32.reshape(NS, ept), dst_pad], axis=1).reshape(NS, nchunks, CC)
    batch32 = batch.astype(jnp.int32)
    fr = frag_levels.reshape(B, 8)
    ad = adduct_feats.reshape(B, 8)

    NP = NS * 640
    deg_call = _build_deg(E, NP)
    scat_call = _build_scatter(NP, HD, nchunks)
    pool_call = _build_pool(N, HD, B)

    degp = deg_call(dst_r32)
    hwp, dis = _k0(x, W_in, degp)

    biases = [b_in.reshape(1, HD)] + [b_mid[i].reshape(1, HD) for i in range(W_mid.shape[0])]
    Ws = [W_mid[i] for i in range(W_mid.shape[0])]
    nlayers = 1 + W_mid.shape[0]

    h = None
    for li in range(nlayers):
        parts = scat_call(hwp.reshape(NC * N, HD // NC), src_r16, dst_r16)
        if li < nlayers - 1:
            hwp = _kc(parts, hwp, dis, biases[li], Ws[li])
        else:
            h = _kc_final(parts, hwp, dis, biases[li])

    gp = pool_call(h, batch32)
    return _head(gp, fr, ad, W_r1, b_r1.reshape(1, HD), W_r2,
                 b_r2.reshape(1, HD + 16), W_out, b_out.reshape(1, W_out.shape[1]))
